# Initial kernel scaffold; baseline (speedup 1.0000x reference)
import jax
import jax.numpy as jnp
from jax.experimental import pallas as pl


def kernel(x, edge_index, weights, W_gcn, W_sage_self, W_sage_neigh, W_gin, W_lin):
    raise NotImplementedError("write your pallas kernel here")


# speedup vs baseline: 19.1100x; 19.1100x over previous
import functools

import jax
import jax.numpy as jnp
from jax import lax
from jax.experimental import pallas as pl
from jax.experimental.pallas import tpu as pltpu
from jax.experimental.pallas import tpu_sc as plsc

N = 10000
E = 320000
D = 128

NC = 2
NS = 16
LANES = 16

EPT_DEG = E // (NC * NS)
K = 100
BLOCKS = E // K
BPT = BLOCKS // NS
NPAD = 10240
RPT = NPAD // NS
ZROWS = 128


def _zero_vmem_1d(ref, n):
    zv = jnp.zeros((LANES,), jnp.float32)

    def body(i, _):
        ref[pl.ds(i * LANES, LANES)] = zv
        return 0

    lax.fori_loop(0, n // LANES, body, 0, unroll=4)


def _zero_vmem_2d(ref, rows, cols):
    zv = jnp.zeros((LANES,), jnp.float32)
    per_row = cols // LANES

    def body(t, _):
        i = t // per_row
        j = t % per_row
        ref[i, pl.ds(j * LANES, LANES)] = zv
        return 0

    lax.fori_loop(0, rows * per_row, body, 0, unroll=4)


def _sc_deg_body(dst_hbm, hists_out, didx, hist):
    c = lax.axis_index("c")
    s = lax.axis_index("s")
    wid = c * NS + s

    _zero_vmem_1d(hist, N)
    pltpu.sync_copy(dst_hbm.at[pl.ds(wid * EPT_DEG, EPT_DEG)], didx)

    ones = jnp.ones((LANES,), jnp.float32)

    def body(j, _):
        idx = didx[pl.ds(j * LANES, LANES)]
        plsc.addupdate_scatter(hist, [idx], ones)
        return 0

    lax.fori_loop(0, EPT_DEG // LANES, body, 0)
    pltpu.sync_copy(hist, hists_out.at[wid])


def _sc_deg(dst):
    mesh = plsc.VectorSubcoreMesh(core_axis_name="c", subcore_axis_name="s")
    f = pl.kernel(
        _sc_deg_body,
        out_type=jax.ShapeDtypeStruct((NC * NS, N), jnp.float32),
        mesh=mesh,
        scratch_types=[
            pltpu.VMEM((EPT_DEG,), jnp.int32),
            pltpu.VMEM((N,), jnp.float32),
        ],
        compiler_params=pltpu.CompilerParams(needs_layout_passes=False),
    )
    return f(dst)


DH = D // 2


def _sc_agg_body(tab_hbm, src0_hbm, src1_hbm, dst_hbm, out,
                 sidx, didx, rows0, rows1, zbuf, acc, sem0, sem1):
    c = lax.axis_index("c")
    s = lax.axis_index("s")

    _zero_vmem_2d(zbuf, ZROWS, DH)
    for b in range(RPT // ZROWS):
        pltpu.sync_copy(zbuf, acc.at[pl.ds(s * RPT + b * ZROWS, ZROWS)])

    base = s * BPT

    @pl.when(c == 0)
    def _():
        pltpu.sync_copy(src0_hbm.at[pl.ds(base, BPT)], sidx)

    @pl.when(c == 1)
    def _():
        pltpu.sync_copy(src1_hbm.at[pl.ds(base, BPT)], sidx)

    pltpu.sync_copy(dst_hbm.at[pl.ds(base, BPT)], didx)
    plsc.subcore_barrier()

    pltpu.make_async_copy(tab_hbm.at[sidx.at[0]], rows0, sem0).start()

    def body(jj, _):
        j0 = 2 * jj
        j1 = j0 + 1
        pltpu.make_async_copy(tab_hbm.at[sidx.at[j1]], rows1, sem1).start()
        pltpu.make_async_copy(tab_hbm.at[sidx.at[j0]], rows0, sem0).wait()
        pltpu.sync_copy(rows0, acc.at[didx.at[j0]], add=True)

        @pl.when(jj < BPT // 2 - 1)
        def _():
            pltpu.make_async_copy(tab_hbm.at[sidx.at[j0 + 2]], rows0,
                                  sem0).start()

        pltpu.make_async_copy(tab_hbm.at[sidx.at[j1]], rows1, sem1).wait()
        pltpu.sync_copy(rows1, acc.at[didx.at[j1]], add=True)
        return 0

    lax.fori_loop(0, BPT // 2, body, 0)

    plsc.subcore_barrier()
    for b in range(RPT // ZROWS):
        r0 = s * RPT + b * ZROWS
        pltpu.sync_copy(acc.at[pl.ds(r0, ZROWS)], zbuf)
        pltpu.sync_copy(zbuf, out.at[c].at[pl.ds(r0, ZROWS)])


def _sc_agg(tab2, src0_rs, src1_rs, dst_rs):
    mesh = plsc.VectorSubcoreMesh(core_axis_name="c", subcore_axis_name="s")
    f = pl.kernel(
        _sc_agg_body,
        out_type=jax.ShapeDtypeStruct((NC, NPAD, DH), jnp.float32),
        mesh=mesh,
        scratch_types=[
            pltpu.VMEM((BPT, K), jnp.int32),
            pltpu.VMEM((BPT, K), jnp.int32),
            pltpu.VMEM((K, DH), jnp.float32),
            pltpu.VMEM((K, DH), jnp.float32),
            pltpu.VMEM((ZROWS, DH), jnp.float32),
            pltpu.VMEM_SHARED((NPAD, DH), jnp.float32),
            pltpu.SemaphoreType.DMA,
            pltpu.SemaphoreType.DMA,
        ],
        compiler_params=pltpu.CompilerParams(needs_layout_passes=False,
                                             use_tc_tiling_on_sc=False),
    )
    return f(tab2, src0_rs, src1_rs, dst_rs)


ROWS_TC = 1000


def _tc_prep_body(hists_ref, x_ref, xs_ref):
    d = jnp.sum(hists_ref[...], axis=1, keepdims=True)
    d = jnp.maximum(d, 1.0)
    xs_ref[...] = lax.rsqrt(d) * x_ref[...]


def _tc_prep(hists_t, x):
    return pl.pallas_call(
        _tc_prep_body,
        grid=(N // ROWS_TC,),
        in_specs=[
            pl.BlockSpec((ROWS_TC, NC * NS), lambda i: (i, 0)),
            pl.BlockSpec((ROWS_TC, D), lambda i: (i, 0)),
        ],
        out_specs=pl.BlockSpec((ROWS_TC, D), lambda i: (i, 0)),
        out_shape=jax.ShapeDtypeStruct((N, D), jnp.float32),
    )(hists_t, x)


def _tc_final_body(w_ref, hists_ref, x_ref, alo_ref, ahi_ref, blo_ref,
                   bhi_ref, wgcn_ref, wss_ref, wsn_ref, wgin_ref, wlin_ref,
                   out_ref):
    w0 = w_ref[0]
    w1 = w_ref[1]
    w2 = w_ref[2]
    w3 = w_ref[3]
    d = jnp.sum(hists_ref[...], axis=1, keepdims=True)
    d = jnp.maximum(d, 1.0)
    r = lax.rsqrt(d)
    inv = 1.0 / d
    wmix = w1 * wss_ref[...] + w2 * wgin_ref[...] + w3 * wlin_ref[...]
    x = x_ref[...]
    a = jnp.concatenate([alo_ref[...], ahi_ref[...]], axis=1)
    b = jnp.concatenate([blo_ref[...], bhi_ref[...]], axis=1)
    acc = jnp.dot(x, wmix, preferred_element_type=jnp.float32)
    acc += w2 * jnp.dot(a, wgin_ref[...], preferred_element_type=jnp.float32)
    acc += (w1 * inv) * jnp.dot(a, wsn_ref[...],
                                preferred_element_type=jnp.float32)
    acc += (w0 * r) * jnp.dot(b, wgcn_ref[...],
                              preferred_element_type=jnp.float32)
    out_ref[...] = acc


def _tc_final(weights, hists_t, x, alo, ahi, blo, bhi,
              wgcn, wss, wsn, wgin, wlin):
    row_spec = pl.BlockSpec((ROWS_TC, D), lambda i: (i, 0))
    half_spec = pl.BlockSpec((ROWS_TC, DH), lambda i: (i, 0))
    w_spec = pl.BlockSpec((D, D), lambda i: (0, 0))
    return pl.pallas_call(
        _tc_final_body,
        grid=(N // ROWS_TC,),
        in_specs=[
            pl.BlockSpec(memory_space=pltpu.SMEM),
            pl.BlockSpec((ROWS_TC, NC * NS), lambda i: (i, 0)),
            row_spec,
            half_spec, half_spec, half_spec, half_spec,
            w_spec, w_spec, w_spec, w_spec, w_spec,
        ],
        out_specs=row_spec,
        out_shape=jax.ShapeDtypeStruct((N, D), jnp.float32),
    )(weights, hists_t, x, alo, ahi, blo, bhi, wgcn, wss, wsn, wgin, wlin)


def kernel(x, edge_index, weights, W_gcn, W_sage_self, W_sage_neigh, W_gin,
           W_lin):
    src = edge_index[0]
    dst = edge_index[1]
    src2 = src * 2
    src0_rs = src2.reshape(BLOCKS, K)
    src1_rs = (src2 + 1).reshape(BLOCKS, K)
    dst_rs = dst.reshape(BLOCKS, K)
    x2 = x.reshape(2 * N, DH)

    hists = _sc_deg(dst)
    hists_t = hists.T
    xs = _tc_prep(hists_t, x)
    xs2 = xs.reshape(2 * N, DH)
    a = _sc_agg(x2, src0_rs, src1_rs, dst_rs)
    b = _sc_agg(xs2, src0_rs, src1_rs, dst_rs)
    return _tc_final(weights, hists_t, x, a[0], a[1], b[0], b[1],
                     W_gcn, W_sage_self, W_sage_neigh, W_gin, W_lin)


# speedup vs baseline: 21.8448x; 1.1431x over previous
import functools

import jax
import jax.numpy as jnp
from jax import lax
from jax.experimental import pallas as pl
from jax.experimental.pallas import tpu as pltpu
from jax.experimental.pallas import tpu_sc as plsc

N = 10000
E = 320000
D = 128

NC = 2
NS = 16
LANES = 16

EPT_DEG = E // (NC * NS)
K = 125
BLOCKS = E // K
BPT = BLOCKS // NS
NPAD = 10240
RPT = NPAD // NS
ZROWS = 128


def _zero_vmem_1d(ref, n):
    zv = jnp.zeros((LANES,), jnp.float32)

    def body(i, _):
        ref[pl.ds(i * LANES, LANES)] = zv
        return 0

    lax.fori_loop(0, n // LANES, body, 0, unroll=4)


def _zero_vmem_2d(ref, rows, cols):
    zv = jnp.zeros((LANES,), jnp.float32)
    per_row = cols // LANES

    def body(t, _):
        i = t // per_row
        j = t % per_row
        ref[i, pl.ds(j * LANES, LANES)] = zv
        return 0

    lax.fori_loop(0, rows * per_row, body, 0, unroll=4)


def _sc_deg_body(dst_hbm, hists_out, didx, hist):
    c = lax.axis_index("c")
    s = lax.axis_index("s")
    wid = c * NS + s

    _zero_vmem_1d(hist, N)
    pltpu.sync_copy(dst_hbm.at[pl.ds(wid * EPT_DEG, EPT_DEG)], didx)

    ones = jnp.ones((LANES,), jnp.float32)

    def body(j, _):
        idx = didx[pl.ds(j * LANES, LANES)]
        plsc.addupdate_scatter(hist, [idx], ones)
        return 0

    lax.fori_loop(0, EPT_DEG // LANES, body, 0)
    pltpu.sync_copy(hist, hists_out.at[wid])


def _sc_deg(dst):
    mesh = plsc.VectorSubcoreMesh(core_axis_name="c", subcore_axis_name="s")
    f = pl.kernel(
        _sc_deg_body,
        out_type=jax.ShapeDtypeStruct((NC * NS, N), jnp.float32),
        mesh=mesh,
        scratch_types=[
            pltpu.VMEM((EPT_DEG,), jnp.int32),
            pltpu.VMEM((N,), jnp.float32),
        ],
        compiler_params=pltpu.CompilerParams(needs_layout_passes=False),
    )
    return f(dst)


DH = D // 2


NBUF = 4


def _sc_agg_body(tab_hbm, src0_hbm, src1_hbm, dst_hbm, out,
                 sidx, didx, r0b, r1b, r2b, r3b, zbuf, acc,
                 g0, g1, g2, g3, s0, s1, s2, s3):
    c = lax.axis_index("c")
    s = lax.axis_index("s")
    bufs = [r0b, r1b, r2b, r3b]
    gsem = [g0, g1, g2, g3]
    ssem = [s0, s1, s2, s3]

    _zero_vmem_2d(zbuf, ZROWS, DH)
    for b in range(RPT // ZROWS):
        pltpu.sync_copy(zbuf, acc.at[pl.ds(s * RPT + b * ZROWS, ZROWS)])

    base = s * BPT

    @pl.when(c == 0)
    def _():
        pltpu.sync_copy(src0_hbm.at[pl.ds(base, BPT)], sidx)

    @pl.when(c == 1)
    def _():
        pltpu.sync_copy(src1_hbm.at[pl.ds(base, BPT)], sidx)

    pltpu.sync_copy(dst_hbm.at[pl.ds(base, BPT)], didx)
    plsc.subcore_barrier()

    def gather(j, b):
        pltpu.make_async_copy(tab_hbm.at[sidx.at[j]], bufs[b], gsem[b]).start()

    def gwait(j, b):
        pltpu.make_async_copy(tab_hbm.at[sidx.at[j]], bufs[b], gsem[b]).wait()

    def scat(j, b):
        pltpu.make_async_copy(bufs[b], acc.at[didx.at[j]],
                              ssem[b]).start(add=True)

    def swait(j, b):
        pltpu.make_async_copy(bufs[b], acc.at[didx.at[j]], ssem[b]).wait()

    gather(0, 0)
    gather(1, 1)

    def body(jj, _):
        for b in range(NBUF):
            j = NBUF * jj + b
            gwait(j, b)
            scat(j, b)
            b2 = (b + 2) % NBUF

            @pl.when(j + 2 < BPT)
            def _():
                @pl.when(j >= 2)
                def _():
                    swait(j - 2, b2)

                gather(j + 2, b2)

        return 0

    lax.fori_loop(0, BPT // NBUF, body, 0)
    for b in range(NBUF):
        swait(BPT - NBUF + b, b)

    plsc.subcore_barrier()
    for b in range(RPT // ZROWS):
        r0 = s * RPT + b * ZROWS
        pltpu.sync_copy(acc.at[pl.ds(r0, ZROWS)], zbuf)
        pltpu.sync_copy(zbuf, out.at[c].at[pl.ds(r0, ZROWS)])


def _sc_agg(tab2, src0_rs, src1_rs, dst_rs):
    mesh = plsc.VectorSubcoreMesh(core_axis_name="c", subcore_axis_name="s")
    f = pl.kernel(
        _sc_agg_body,
        out_type=jax.ShapeDtypeStruct((NC, NPAD, DH), jnp.float32),
        mesh=mesh,
        scratch_types=[
            pltpu.VMEM((BPT, K), jnp.int32),
            pltpu.VMEM((BPT, K), jnp.int32),
            pltpu.VMEM((K, DH), jnp.float32),
            pltpu.VMEM((K, DH), jnp.float32),
            pltpu.VMEM((K, DH), jnp.float32),
            pltpu.VMEM((K, DH), jnp.float32),
            pltpu.VMEM((ZROWS, DH), jnp.float32),
            pltpu.VMEM_SHARED((NPAD, DH), jnp.float32),
            pltpu.SemaphoreType.DMA, pltpu.SemaphoreType.DMA,
            pltpu.SemaphoreType.DMA, pltpu.SemaphoreType.DMA,
            pltpu.SemaphoreType.DMA, pltpu.SemaphoreType.DMA,
            pltpu.SemaphoreType.DMA, pltpu.SemaphoreType.DMA,
        ],
        compiler_params=pltpu.CompilerParams(needs_layout_passes=False,
                                             use_tc_tiling_on_sc=False),
    )
    return f(tab2, src0_rs, src1_rs, dst_rs)


ROWS_TC = 1000


def _tc_prep_body(hists_ref, x_ref, xs_ref):
    d = jnp.sum(hists_ref[...], axis=1, keepdims=True)
    d = jnp.maximum(d, 1.0)
    xs_ref[...] = lax.rsqrt(d) * x_ref[...]


def _tc_prep(hists_t, x):
    return pl.pallas_call(
        _tc_prep_body,
        grid=(N // ROWS_TC,),
        in_specs=[
            pl.BlockSpec((ROWS_TC, NC * NS), lambda i: (i, 0)),
            pl.BlockSpec((ROWS_TC, D), lambda i: (i, 0)),
        ],
        out_specs=pl.BlockSpec((ROWS_TC, D), lambda i: (i, 0)),
        out_shape=jax.ShapeDtypeStruct((N, D), jnp.float32),
    )(hists_t, x)


def _tc_final_body(w_ref, hists_ref, x_ref, alo_ref, ahi_ref, blo_ref,
                   bhi_ref, wgcn_ref, wss_ref, wsn_ref, wgin_ref, wlin_ref,
                   out_ref):
    w0 = w_ref[0]
    w1 = w_ref[1]
    w2 = w_ref[2]
    w3 = w_ref[3]
    d = jnp.sum(hists_ref[...], axis=1, keepdims=True)
    d = jnp.maximum(d, 1.0)
    r = lax.rsqrt(d)
    inv = 1.0 / d
    wmix = w1 * wss_ref[...] + w2 * wgin_ref[...] + w3 * wlin_ref[...]
    x = x_ref[...]
    a = jnp.concatenate([alo_ref[...], ahi_ref[...]], axis=1)
    b = jnp.concatenate([blo_ref[...], bhi_ref[...]], axis=1)
    acc = jnp.dot(x, wmix, preferred_element_type=jnp.float32)
    acc += w2 * jnp.dot(a, wgin_ref[...], preferred_element_type=jnp.float32)
    acc += (w1 * inv) * jnp.dot(a, wsn_ref[...],
                                preferred_element_type=jnp.float32)
    acc += (w0 * r) * jnp.dot(b, wgcn_ref[...],
                              preferred_element_type=jnp.float32)
    out_ref[...] = acc


def _tc_final(weights, hists_t, x, alo, ahi, blo, bhi,
              wgcn, wss, wsn, wgin, wlin):
    row_spec = pl.BlockSpec((ROWS_TC, D), lambda i: (i, 0))
    half_spec = pl.BlockSpec((ROWS_TC, DH), lambda i: (i, 0))
    w_spec = pl.BlockSpec((D, D), lambda i: (0, 0))
    return pl.pallas_call(
        _tc_final_body,
        grid=(N // ROWS_TC,),
        in_specs=[
            pl.BlockSpec(memory_space=pltpu.SMEM),
            pl.BlockSpec((ROWS_TC, NC * NS), lambda i: (i, 0)),
            row_spec,
            half_spec, half_spec, half_spec, half_spec,
            w_spec, w_spec, w_spec, w_spec, w_spec,
        ],
        out_specs=row_spec,
        out_shape=jax.ShapeDtypeStruct((N, D), jnp.float32),
    )(weights, hists_t, x, alo, ahi, blo, bhi, wgcn, wss, wsn, wgin, wlin)


def kernel(x, edge_index, weights, W_gcn, W_sage_self, W_sage_neigh, W_gin,
           W_lin):
    src = edge_index[0]
    dst = edge_index[1]
    src2 = src * 2
    src0_rs = src2.reshape(BLOCKS, K)
    src1_rs = (src2 + 1).reshape(BLOCKS, K)
    dst_rs = dst.reshape(BLOCKS, K)
    x2 = x.reshape(2 * N, DH)

    hists = _sc_deg(dst)
    hists_t = hists.T
    xs = _tc_prep(hists_t, x)
    xs2 = xs.reshape(2 * N, DH)
    a = _sc_agg(x2, src0_rs, src1_rs, dst_rs)
    b = _sc_agg(xs2, src0_rs, src1_rs, dst_rs)
    return _tc_final(weights, hists_t, x, a[0], a[1], b[0], b[1],
                     W_gcn, W_sage_self, W_sage_neigh, W_gin, W_lin)
